# Initial kernel scaffold; baseline (speedup 1.0000x reference)
import functools

import numpy as np
import jax
import jax.numpy as jnp
from jax import lax
from jax.experimental import pallas as pl
from jax.experimental.pallas import tpu as pltpu
from jax.experimental.pallas import tpu_sc as plsc

_L = 8
_T = 2 ** 19
_N_MIN = 32
_B_GROWTH = float(np.exp(np.log(2048.0 / _N_MIN) / (_L - 1)))
_RES = [int(np.floor(_N_MIN * (_B_GROWTH ** l))) for l in range(_L)]
_TSIZES = [min(_T, (r + 1) ** 3) for r in _RES]
_DENSE = [(r + 1) ** 3 <= _T for r in _RES]
_N = 262144

_NC, _NS = 2, 16
_NW = _NC * _NS
_PW = _N // _NW
_P = 2048
_NSUB = _PW // _P

_H2 = np.uint32(2654435761)
_H3 = np.uint32(805459861)


def _encode_body(xT, *rest):
    tabs = rest[:_L]
    out = rest[_L]
    xbuf = rest[_L + 1]
    ibufs = rest[_L + 2:_L + 10]
    wbufs = rest[_L + 10:_L + 18]
    rbufs = rest[_L + 18:_L + 26]
    enc16 = rest[_L + 26]
    sem = rest[_L + 27]

    wid = lax.axis_index("s") * _NC + lax.axis_index("c")
    lanes = lax.iota(jnp.int32, 16)
    half = lanes // 2
    fcol = lanes % 2
    pat = half * 16 + fcol

    def subchunk(s, carry):
        base = wid * _PW + s * _P
        pltpu.sync_copy(xT.at[:, pl.ds(base, _P)], xbuf)

        for l in range(_L):
            res = _RES[l]
            resf = float(res)

            def pass_a(g, c2, l=l, res=res, resf=resf):
                o = g * 16
                ps, ws = [], []
                for d in range(3):
                    v = xbuf[d, pl.ds(o, 16)]
                    xn = (v + 1.0) * 0.5
                    pos = xn * resf
                    p0 = pos.astype(jnp.int32)
                    w = pos - p0.astype(jnp.float32)
                    ps.append(p0)
                    ws.append(w)
                px, py, pz = ps
                wx, wy, wz = ws
                if _DENSE[l]:
                    r1 = res + 1
                    ax = (px, px + 1)
                    ay = (py * r1, (py + 1) * r1)
                    az = (pz * (r1 * r1), (pz + 1) * (r1 * r1))
                else:
                    pxu = px.astype(jnp.uint32)
                    pyu = py.astype(jnp.uint32)
                    pzu = pz.astype(jnp.uint32)
                    one = np.uint32(1)
                    ax = (pxu, pxu + one)
                    ay = (pyu * _H2, (pyu + one) * _H2)
                    az = (pzu * _H3, (pzu + one) * _H3)
                wxs = (1.0 - wx, wx)
                wys = (1.0 - wy, wy)
                wzs = (1.0 - wz, wz)
                mask = np.uint32(_TSIZES[l] - 1)
                for dz in (0, 1):
                    for dy in (0, 1):
                        pxy0 = wxs[0] * wys[dy]
                        pxy1 = wxs[1] * wys[dy]
                        for dx in (0, 1):
                            c = dz * 4 + dy * 2 + dx
                            if _DENSE[l]:
                                idx = ax[dx] + ay[dy] + az[dz]
                            else:
                                h = ax[dx] ^ ay[dy] ^ az[dz]
                                idx = (h & mask).astype(jnp.int32)
                            wc = (pxy1 if dx else pxy0) * wzs[dz]
                            ibufs[c][pl.ds(o, 16)] = idx
                            wbufs[c][pl.ds(o, 16)] = wc
                return c2

            lax.fori_loop(0, _P // 16, pass_a, 0)

            descs = [pltpu.async_copy(tabs[l].at[ibufs[c]], rbufs[c], sem)
                     for c in range(8)]
            for dsc in descs:
                dsc.wait()

            def pass_b(g, c2, l=l):
                rowi = half + g * 8
                acc = jnp.zeros((16,), jnp.float32)
                for c in range(8):
                    rv = plsc.load_gather(rbufs[c], [rowi, fcol])
                    wv = plsc.load_gather(wbufs[c], [rowi])
                    acc = acc + wv * rv
                plsc.store_scatter(enc16, [pat + (g * 128 + 2 * l)], acc)
                return c2

            lax.fori_loop(0, _P // 8, pass_b, 0)

        pltpu.sync_copy(enc16, out.at[pl.ds(base * 16, _P * 16)])
        return carry

    lax.fori_loop(0, _NSUB, subchunk, 0)


_encode = functools.partial(
    pl.kernel,
    out_type=jax.ShapeDtypeStruct((_N * 16,), jnp.float32),
    mesh=plsc.VectorSubcoreMesh(core_axis_name="c", subcore_axis_name="s",
                                num_cores=_NC, num_subcores=_NS),
    scratch_types=(
        [pltpu.VMEM((3, _P), jnp.float32)]
        + [pltpu.VMEM((_P,), jnp.int32) for _ in range(8)]
        + [pltpu.VMEM((_P,), jnp.float32) for _ in range(8)]
        + [pltpu.VMEM((_P, 2), jnp.float32) for _ in range(8)]
        + [pltpu.VMEM((_P * 16,), jnp.float32),
           pltpu.SemaphoreType.DMA]
    ),
)(_encode_body)


_NB = 4096


def _mlp_body(enc_ref, w1_ref, b1_ref, w2_ref, b2_ref, out_ref):
    e = enc_ref[...]
    h = jnp.dot(e, w1_ref[...], preferred_element_type=jnp.float32)
    h = jnp.maximum(h + b1_ref[...], 0.0)
    v = jnp.dot(h, w2_ref[...], preferred_element_type=jnp.float32)
    v = v + b2_ref[...]
    out_ref[...] = 1.0 / (1.0 + jnp.exp(-v))


_mlp = pl.pallas_call(
    _mlp_body,
    grid=(_N // _NB,),
    in_specs=[
        pl.BlockSpec((_NB, 16), lambda i: (i, 0)),
        pl.BlockSpec((16, 32), lambda i: (0, 0)),
        pl.BlockSpec((32,), lambda i: (0,)),
        pl.BlockSpec((32, 1), lambda i: (0, 0)),
        pl.BlockSpec((1,), lambda i: (0,)),
    ],
    out_specs=pl.BlockSpec((_NB, 1), lambda i: (i, 0)),
    out_shape=jax.ShapeDtypeStruct((_N, 1), jnp.float32),
)


def kernel(x, tables, W1, b1, W2, b2):
    xT = x.T
    enc = _encode(xT, *tables)
    vis = _mlp(enc.reshape(_N, 16), W1, b1, W2, b2)
    return vis.reshape(_N)


# speedup vs baseline: 3.0634x; 3.0634x over previous
import functools

import numpy as np
import jax
import jax.numpy as jnp
from jax import lax
from jax.experimental import pallas as pl
from jax.experimental.pallas import tpu as pltpu
from jax.experimental.pallas import tpu_sc as plsc

_L = 8
_T = 2 ** 19
_N_MIN = 32
_B_GROWTH = float(np.exp(np.log(2048.0 / _N_MIN) / (_L - 1)))
_RES = [int(np.floor(_N_MIN * (_B_GROWTH ** l))) for l in range(_L)]
_TSIZES = [min(_T, (r + 1) ** 3) for r in _RES]
_DENSE = [(r + 1) ** 3 <= _T for r in _RES]
_N = 262144

_NC, _NS = 2, 16
_NW = _NC * _NS
_PW = _N // _NW
_P = 512
_NSUB = _PW // _P

_H2 = np.uint32(2654435761)
_H3 = np.uint32(805459861)


def _encode_body(xT, *rest):
    tabs = rest[:_L]
    out = rest[_L]
    r = _L + 1
    xbuf = rest[r]; r += 1
    ibanks = (rest[r:r + 16], rest[r + 16:r + 32]); r += 32
    wbanks = (rest[r:r + 8], rest[r + 8:r + 16]); r += 16
    rbanks = (rest[r:r + 16], rest[r + 16:r + 32]); r += 32
    enc16t = rest[r]; r += 1
    sems = (rest[r], rest[r + 1])

    wid = lax.axis_index("s") * _NC + lax.axis_index("c")

    def make_pass_a(l, bank):
        res = _RES[l]
        resf = float(res)
        ibufs, wbufs = ibanks[bank], wbanks[bank]

        def pass_a(g, c2):
            o = g * 16
            ps, ws = [], []
            for d in range(3):
                v = xbuf[d, pl.ds(o, 16)]
                xn = (v + 1.0) * 0.5
                pos = xn * resf
                p0 = pos.astype(jnp.int32)
                w = pos - p0.astype(jnp.float32)
                ps.append(p0)
                ws.append(w)
            px, py, pz = ps
            wx, wy, wz = ws
            if _DENSE[l]:
                r1 = res + 1
                ax = (px, px + 1)
                ay = (py * r1, (py + 1) * r1)
                az = (pz * (r1 * r1), (pz + 1) * (r1 * r1))
            else:
                pxu = px.astype(jnp.uint32)
                pyu = py.astype(jnp.uint32)
                pzu = pz.astype(jnp.uint32)
                one = np.uint32(1)
                ax = (pxu, pxu + one)
                ay = (pyu * _H2, (pyu + one) * _H2)
                az = (pzu * _H3, (pzu + one) * _H3)
            wxs = (1.0 - wx, wx)
            wys = (1.0 - wy, wy)
            wzs = (1.0 - wz, wz)
            mask = np.uint32(_TSIZES[l] - 1)
            for dz in (0, 1):
                for dy in (0, 1):
                    pxy0 = wxs[0] * wys[dy]
                    pxy1 = wxs[1] * wys[dy]
                    for dx in (0, 1):
                        c = dz * 4 + dy * 2 + dx
                        if _DENSE[l]:
                            idx = ax[dx] + ay[dy] + az[dz]
                        else:
                            h = ax[dx] ^ ay[dy] ^ az[dz]
                            idx = (h & mask).astype(jnp.int32)
                        wc = (pxy1 if dx else pxy0) * wzs[dz]
                        i2 = idx + idx
                        ibufs[2 * c][pl.ds(o, 16)] = i2
                        ibufs[2 * c + 1][pl.ds(o, 16)] = i2 + 1
                        wbufs[c][pl.ds(o, 16)] = wc
            return c2

        return pass_a

    def make_pass_b(l, bank):
        wbufs, rbufs = wbanks[bank], rbanks[bank]

        def pass_b(g, c2):
            o = g * 16
            acc0 = jnp.zeros((16,), jnp.float32)
            acc1 = jnp.zeros((16,), jnp.float32)
            for c in range(8):
                wv = wbufs[c][pl.ds(o, 16)]
                acc0 = acc0 + wv * rbufs[2 * c][pl.ds(o, 16)]
                acc1 = acc1 + wv * rbufs[2 * c + 1][pl.ds(o, 16)]
            enc16t[2 * l, pl.ds(o, 16)] = acc0
            enc16t[2 * l + 1, pl.ds(o, 16)] = acc1
            return c2

        return pass_b

    def subchunk(s, carry):
        base = wid * _PW + s * _P
        pltpu.sync_copy(xT.at[:, pl.ds(base, _P)], xbuf)

        prev = None
        for l in range(_L):
            b = l & 1
            lax.fori_loop(0, _P // 16, make_pass_a(l, b), 0)
            descs = [pltpu.async_copy(tabs[l].at[ibanks[b][j]],
                                      rbanks[b][j], sems[b])
                     for j in range(16)]
            if prev is not None:
                for dsc in prev:
                    dsc.wait()
                lax.fori_loop(0, _P // 16, make_pass_b(l - 1, 1 - b), 0)
            prev = descs
        for dsc in prev:
            dsc.wait()
        lax.fori_loop(0, _P // 16, make_pass_b(_L - 1, 1), 0)

        pltpu.sync_copy(enc16t, out.at[:, pl.ds(base, _P)])
        return carry

    lax.fori_loop(0, _NSUB, subchunk, 0)


@functools.cache
def _get_encode():
    return pl.kernel(
        _encode_body,
        out_type=jax.ShapeDtypeStruct((16, _N), jnp.float32),
        mesh=plsc.VectorSubcoreMesh(core_axis_name="c", subcore_axis_name="s",
                                    num_cores=_NC, num_subcores=_NS),
        compiler_params=pltpu.CompilerParams(use_tc_tiling_on_sc=False,
                                             needs_layout_passes=False),
        scratch_types=(
            [pltpu.VMEM((3, _P), jnp.float32)]
            + [pltpu.VMEM((_P,), jnp.int32) for _ in range(32)]
            + [pltpu.VMEM((_P,), jnp.float32) for _ in range(16)]
            + [pltpu.VMEM((_P,), jnp.float32) for _ in range(32)]
            + [pltpu.VMEM((16, _P), jnp.float32),
               pltpu.SemaphoreType.DMA, pltpu.SemaphoreType.DMA]
        ),
    )


_NB = 4096


def _mlp_body(et_ref, w1t_ref, b1_ref, w2t_ref, b2_ref, out_ref):
    et = et_ref[...]
    h = jnp.dot(w1t_ref[...], et, preferred_element_type=jnp.float32)
    h = jnp.maximum(h + b1_ref[...], 0.0)
    v = jnp.dot(w2t_ref[...], h, preferred_element_type=jnp.float32)
    v = v + b2_ref[...]
    out_ref[...] = 1.0 / (1.0 + jnp.exp(-v))


_mlp = pl.pallas_call(
    _mlp_body,
    grid=(_N // _NB,),
    in_specs=[
        pl.BlockSpec((16, _NB), lambda i: (0, i)),
        pl.BlockSpec((32, 16), lambda i: (0, 0)),
        pl.BlockSpec((32, 1), lambda i: (0, 0)),
        pl.BlockSpec((1, 32), lambda i: (0, 0)),
        pl.BlockSpec((1,), lambda i: (0,)),
    ],
    out_specs=pl.BlockSpec((1, _NB), lambda i: (0, i)),
    out_shape=jax.ShapeDtypeStruct((1, _N), jnp.float32),
)


def kernel(x, tables, W1, b1, W2, b2):
    xT = x.T
    tflat = [t.reshape(-1) for t in tables]
    encT = _get_encode()(xT, *tflat)
    vis = _mlp(encT, W1.T, b1.reshape(32, 1), W2.T, b2)
    return vis.reshape(_N)


# speedup vs baseline: 6.9426x; 2.2663x over previous
import functools

import numpy as np
import jax
import jax.numpy as jnp
from jax import lax
from jax.experimental import pallas as pl
from jax.experimental.pallas import tpu as pltpu
from jax.experimental.pallas import tpu_sc as plsc

_L = 8
_T = 2 ** 19
_N_MIN = 32
_B_GROWTH = float(np.exp(np.log(2048.0 / _N_MIN) / (_L - 1)))
_RES = [int(np.floor(_N_MIN * (_B_GROWTH ** l))) for l in range(_L)]
_TSIZES = [min(_T, (r + 1) ** 3) for r in _RES]
_DENSE = [(r + 1) ** 3 <= _T for r in _RES]
_N = 262144

_NC, _NS = 2, 16
_NW = _NC * _NS
_PW = _N // _NW
_P = 512
_NSUB = _PW // _P

_H2 = np.uint32(2654435761)
_H3 = np.uint32(805459861)


def _encode_body(xT, *rest):
    tabs = rest[:_L]
    out = rest[_L]
    r = _L + 1
    xbuf = rest[r]; r += 1
    ibanks = (rest[r:r + 16], rest[r + 16:r + 32]); r += 32
    wbanks = (rest[r:r + 8], rest[r + 8:r + 16]); r += 16
    rbanks = (rest[r:r + 16], rest[r + 16:r + 32]); r += 32
    enc16t = rest[r]; r += 1
    sems = (rest[r], rest[r + 1])

    wid = lax.axis_index("s") * _NC + lax.axis_index("c")

    def make_pass_a(l, bank):
        res = _RES[l]
        resf = float(res)
        ibufs, wbufs = ibanks[bank], wbanks[bank]

        def pass_a(g, c2):
            o = g * 16
            ps, ws = [], []
            for d in range(3):
                v = xbuf[d, pl.ds(o, 16)]
                xn = (v + 1.0) * 0.5
                pos = xn * resf
                p0 = pos.astype(jnp.int32)
                w = pos - p0.astype(jnp.float32)
                ps.append(p0)
                ws.append(w)
            px, py, pz = ps
            wx, wy, wz = ws
            if _DENSE[l]:
                r1 = res + 1
                ax = (px, px + 1)
                ay = (py * r1, (py + 1) * r1)
                az = (pz * (r1 * r1), (pz + 1) * (r1 * r1))
            else:
                pxu = px.astype(jnp.uint32)
                pyu = py.astype(jnp.uint32)
                pzu = pz.astype(jnp.uint32)
                one = np.uint32(1)
                ax = (pxu, pxu + one)
                ay = (pyu * _H2, (pyu + one) * _H2)
                az = (pzu * _H3, (pzu + one) * _H3)
            wxs = (1.0 - wx, wx)
            wys = (1.0 - wy, wy)
            wzs = (1.0 - wz, wz)
            mask = np.uint32(_TSIZES[l] - 1)
            for dz in (0, 1):
                for dy in (0, 1):
                    pxy0 = wxs[0] * wys[dy]
                    pxy1 = wxs[1] * wys[dy]
                    for dx in (0, 1):
                        c = dz * 4 + dy * 2 + dx
                        if _DENSE[l]:
                            idx = ax[dx] + ay[dy] + az[dz]
                        else:
                            h = ax[dx] ^ ay[dy] ^ az[dz]
                            idx = (h & mask).astype(jnp.int32)
                        wc = (pxy1 if dx else pxy0) * wzs[dz]
                        p0 = idx + (idx & -128)
                        ibufs[2 * c][pl.ds(o, 16)] = p0
                        ibufs[2 * c + 1][pl.ds(o, 16)] = p0 + 128
                        wbufs[c][pl.ds(o, 16)] = wc
            return c2

        return pass_a

    def make_pass_b(l, bank):
        wbufs, rbufs = wbanks[bank], rbanks[bank]

        def pass_b(g, c2):
            o = g * 16
            acc0 = jnp.zeros((16,), jnp.float32)
            acc1 = jnp.zeros((16,), jnp.float32)
            for c in range(8):
                wv = wbufs[c][pl.ds(o, 16)]
                acc0 = acc0 + wv * rbufs[2 * c][pl.ds(o, 16)]
                acc1 = acc1 + wv * rbufs[2 * c + 1][pl.ds(o, 16)]
            enc16t[2 * l, pl.ds(o, 16)] = acc0
            enc16t[2 * l + 1, pl.ds(o, 16)] = acc1
            return c2

        return pass_b

    def subchunk(s, carry):
        base = wid * _PW + s * _P
        pltpu.sync_copy(xT.at[:, pl.ds(base, _P)], xbuf)

        prev = None
        for l in range(_L):
            b = l & 1
            lax.fori_loop(0, _P // 16, make_pass_a(l, b), 0)
            descs = [pltpu.async_copy(tabs[l].at[ibanks[b][j]],
                                      rbanks[b][j], sems[b])
                     for j in range(16)]
            if prev is not None:
                for dsc in prev:
                    dsc.wait()
                lax.fori_loop(0, _P // 16, make_pass_b(l - 1, 1 - b), 0)
            prev = descs
        for dsc in prev:
            dsc.wait()
        lax.fori_loop(0, _P // 16, make_pass_b(_L - 1, 1), 0)

        pltpu.sync_copy(enc16t, out.at[:, pl.ds(base, _P)])
        return carry

    lax.fori_loop(0, _NSUB, subchunk, 0)


@functools.cache
def _get_encode():
    return pl.kernel(
        _encode_body,
        out_type=jax.ShapeDtypeStruct((16, _N), jnp.float32),
        mesh=plsc.VectorSubcoreMesh(core_axis_name="c", subcore_axis_name="s",
                                    num_cores=_NC, num_subcores=_NS),
        compiler_params=pltpu.CompilerParams(use_tc_tiling_on_sc=False,
                                             needs_layout_passes=False),
        scratch_types=(
            [pltpu.VMEM((3, _P), jnp.float32)]
            + [pltpu.VMEM((_P,), jnp.int32) for _ in range(32)]
            + [pltpu.VMEM((_P,), jnp.float32) for _ in range(16)]
            + [pltpu.VMEM((_P,), jnp.float32) for _ in range(32)]
            + [pltpu.VMEM((16, _P), jnp.float32),
               pltpu.SemaphoreType.DMA, pltpu.SemaphoreType.DMA]
        ),
    )


_NB = 4096


def _mlp_body(et_ref, w1t_ref, b1_ref, w2t_ref, b2_ref, out_ref):
    et = et_ref[...]
    h = jnp.dot(w1t_ref[...], et, preferred_element_type=jnp.float32)
    h = jnp.maximum(h + b1_ref[...], 0.0)
    v = jnp.dot(w2t_ref[...], h, preferred_element_type=jnp.float32)
    v = v + b2_ref[...]
    out_ref[...] = 1.0 / (1.0 + jnp.exp(-v))


_mlp = pl.pallas_call(
    _mlp_body,
    grid=(_N // _NB,),
    in_specs=[
        pl.BlockSpec((16, _NB), lambda i: (0, i)),
        pl.BlockSpec((32, 16), lambda i: (0, 0)),
        pl.BlockSpec((32, 1), lambda i: (0, 0)),
        pl.BlockSpec((1, 32), lambda i: (0, 0)),
        pl.BlockSpec((1,), lambda i: (0,)),
    ],
    out_specs=pl.BlockSpec((1, _NB), lambda i: (0, i)),
    out_shape=jax.ShapeDtypeStruct((1, _N), jnp.float32),
)


def _tile_view(t):
    rows = t.shape[0]
    pad = (-rows) % 128
    if pad:
        t = jnp.pad(t, ((0, pad), (0, 0)))
    return t.reshape(-1, 128, 2).transpose(0, 2, 1).reshape(-1)


def kernel(x, tables, W1, b1, W2, b2):
    xT = x.T
    tflat = [_tile_view(t) for t in tables]
    encT = _get_encode()(xT, *tflat)
    vis = _mlp(encT, W1.T, b1.reshape(32, 1), W2.T, b2)
    return vis.reshape(_N)
